# baseline (speedup 1.0000x reference)
import functools

import jax
import jax.numpy as jnp
from jax import lax
from jax.experimental import pallas as pl
from jax.experimental.pallas import tpu as pltpu
from jax.experimental.pallas import tpu_sc as plsc

N = 10000
E = 320000
NC = 2
NS = 16
NW = NC * NS
CHUNK = 128
NCH_FULL = 80
NCH_LAST = 20
ROWS_PER_TILE = N // NS
BN = 400
NBUF = 5
LOOK = NBUF - 2


def _make_sc_aggregate(D):
    mesh = plsc.VectorSubcoreMesh(core_axis_name="c", subcore_axis_name="s")
    scratch = (
        [pltpu.VMEM((NCH_FULL, CHUNK), jnp.int32),
         pltpu.VMEM((NCH_FULL, CHUNK), jnp.int32),
         pltpu.VMEM((NBUF, CHUNK, D), jnp.float32),
         pltpu.VMEM_SHARED((N, D), jnp.float32)]
        + [pltpu.SemaphoreType.DMA] * (2 * NBUF)
    )

    @functools.partial(
        pl.kernel, mesh=mesh,
        out_type=jax.ShapeDtypeStruct((NC, N, D), jnp.float32),
        scratch_types=scratch,
        compiler_params=pltpu.CompilerParams(use_tc_tiling_on_sc=False))
    def agg(vals, src3, dst3, out, src_v, dst_v, rows, acc, *sems):
        gsem = sems[:NBUF]
        ssem = sems[NBUF:]
        c = lax.axis_index("c")
        s = lax.axis_index("s")
        wid = c * NS + s
        r0 = s * ROWS_PER_TILE
        nch = jnp.where(wid == NW - 1, NCH_LAST, NCH_FULL)

        zero16 = jnp.zeros((16,), jnp.float32)

        def zrow(r, _):
            def zcol(k, _):
                rows[0, r, pl.ds(k * 16, 16)] = zero16
                return 0
            return lax.fori_loop(0, D // 16, zcol, 0)
        lax.fori_loop(0, CHUNK, zrow, 0)
        for b in range(ROWS_PER_TILE // CHUNK):
            pltpu.sync_copy(rows.at[0], acc.at[pl.ds(r0 + b * CHUNK, CHUNK)])
        rem = ROWS_PER_TILE % CHUNK
        base = ROWS_PER_TILE - rem
        pltpu.sync_copy(rows.at[0, pl.ds(0, rem)],
                        acc.at[pl.ds(r0 + base, rem)])

        @pl.when(wid == NW - 1)
        def _():
            pltpu.sync_copy(src3.at[pl.ds((NW - 1) * NCH_FULL, NCH_LAST)],
                            src_v.at[pl.ds(0, NCH_LAST)])
            pltpu.sync_copy(dst3.at[pl.ds((NW - 1) * NCH_FULL, NCH_LAST)],
                            dst_v.at[pl.ds(0, NCH_LAST)])

        @pl.when(wid != NW - 1)
        def _():
            pltpu.sync_copy(src3.at[pl.ds(wid * NCH_FULL, NCH_FULL)], src_v)
            pltpu.sync_copy(dst3.at[pl.ds(wid * NCH_FULL, NCH_FULL)], dst_v)

        plsc.subcore_barrier()

        def fire_g(j, b):
            pltpu.async_copy(vals.at[src_v.at[j]], rows.at[b], gsem[b])

        def wait_g(j, b):
            pltpu.make_async_copy(vals.at[src_v.at[j]], rows.at[b],
                                  gsem[b]).wait()

        def fire_s(j, b):
            pltpu.async_copy(rows.at[b], acc.at[dst_v.at[j]], ssem[b],
                             add=True)

        def wait_s(j, b):
            pltpu.make_async_copy(rows.at[b], acc.at[dst_v.at[j]],
                                  ssem[b]).wait()

        for j in range(LOOK):
            fire_g(j, j)
        for j in range(NBUF):
            if j - 2 >= 0:
                wait_s(j - 2, (j - 2) % NBUF)
            fire_g(j + LOOK, (j + LOOK) % NBUF)
            wait_g(j, j)
            fire_s(j, j)

        def body(i, _):
            j0 = i * NBUF
            for b in range(NBUF):
                j = j0 + b
                wait_s(j - 2, (b - 2) % NBUF)
                fire_g(j + LOOK, (b + LOOK) % NBUF)
                wait_g(j, b)
                fire_s(j, b)
            return 0
        lax.fori_loop(1, nch // NBUF - 1, body, 0)

        for k in range(NBUF):
            j = nch - NBUF + k
            wait_s(j - 2, (k - 2) % NBUF)
            if k + LOOK < NBUF:
                fire_g(j + LOOK, (k + LOOK) % NBUF)
            wait_g(j, k)
            fire_s(j, k)
        for k in range(NBUF - 2, NBUF):
            wait_s(nch - NBUF + k, k)

        plsc.subcore_barrier()

        pltpu.sync_copy(acc.at[pl.ds(r0, ROWS_PER_TILE)],
                        out.at[c, pl.ds(r0, ROWS_PER_TILE)])

    return agg


_sc_agg_80 = _make_sc_aggregate(80)
_sc_agg_64 = _make_sc_aggregate(64)


def _tc_xr_body(x_ref, w1rt_ref, b1_ref, xr_ref):
    xr_ref[...] = x_ref[...] @ w1rt_ref[...] + b1_ref[...]


def _tc_layer1_body(acca_ref, accb_ref, xr_ref, w1lta_ref, w1ltb_ref,
                    w2lt_ref, y2_ref, h_ref, deg_ref):
    a = acca_ref[0] + acca_ref[1]
    b = accb_ref[0] + accb_ref[1]
    deg = jnp.maximum(b[:, 48], 1.0)
    lin = a @ w1lta_ref[...] + b[:, :48] @ w1ltb_ref[...]
    h = jnp.maximum(lin / deg[:, None] + xr_ref[...], 0.0)
    h_ref[...] = h
    y2_ref[...] = h @ w2lt_ref[...]
    deg_ref[...] = jnp.broadcast_to(deg[:, None], deg_ref.shape)


def _tc_hr_body(h_ref, w2rt_ref, b2_ref, hr_ref):
    hr_ref[...] = h_ref[...] @ w2rt_ref[...] + b2_ref[...]


def _tc_layer2_body(acc_ref, deg_ref, hr_ref, o_ref):
    a = acc_ref[0] + acc_ref[1]
    v = a / deg_ref[:, :1] + hr_ref[...]
    z = v - jnp.max(v, axis=-1, keepdims=True)
    o_ref[...] = z - jnp.log(jnp.sum(jnp.exp(z), axis=-1, keepdims=True))


def kernel(x, edge_index, W1_l, b1, W1_r, W2_l, b2, W2_r):
    ei = edge_index.astype(jnp.int32)
    src3 = ei[0].reshape(E // CHUNK, CHUNK)
    dst3 = ei[1].reshape(E // CHUNK, CHUNK)

    xa = x[:, :80]
    xb = jnp.concatenate(
        [x[:, 80:], jnp.ones((N, 1), jnp.float32),
         jnp.zeros((N, 15), jnp.float32)], axis=1)

    grid = (N // BN,)
    wfull = lambda shp: pl.BlockSpec(shp, lambda i: (0, 0))

    xr = pl.pallas_call(
        _tc_xr_body,
        grid=grid,
        in_specs=[pl.BlockSpec((BN, 128), lambda i: (i, 0)),
                  wfull((128, 128)), wfull((1, 128))],
        out_specs=pl.BlockSpec((BN, 128), lambda i: (i, 0)),
        out_shape=jax.ShapeDtypeStruct((N, 128), jnp.float32),
    )(x, W1_r.T, b1[None, :])

    acca = _sc_agg_80(xa, src3, dst3)
    accb = _sc_agg_64(xb, src3, dst3)

    y2, h, deg8 = pl.pallas_call(
        _tc_layer1_body,
        grid=grid,
        in_specs=[
            pl.BlockSpec((NC, BN, 80), lambda i: (0, i, 0)),
            pl.BlockSpec((NC, BN, 64), lambda i: (0, i, 0)),
            pl.BlockSpec((BN, 128), lambda i: (i, 0)),
            wfull((80, 128)),
            wfull((48, 128)),
            wfull((128, 64)),
        ],
        out_specs=[pl.BlockSpec((BN, 64), lambda i: (i, 0)),
                   pl.BlockSpec((BN, 128), lambda i: (i, 0)),
                   pl.BlockSpec((BN, 8), lambda i: (i, 0))],
        out_shape=[jax.ShapeDtypeStruct((N, 64), jnp.float32),
                   jax.ShapeDtypeStruct((N, 128), jnp.float32),
                   jax.ShapeDtypeStruct((N, 8), jnp.float32)],
    )(acca, accb, xr, W1_l.T[:80], W1_l.T[80:], W2_l.T)

    acc2 = _sc_agg_64(y2, src3, dst3)

    hr = pl.pallas_call(
        _tc_hr_body,
        grid=grid,
        in_specs=[pl.BlockSpec((BN, 128), lambda i: (i, 0)),
                  wfull((128, 64)), wfull((1, 64))],
        out_specs=pl.BlockSpec((BN, 64), lambda i: (i, 0)),
        out_shape=jax.ShapeDtypeStruct((N, 64), jnp.float32),
    )(h, W2_r.T, b2[None, :])

    out = pl.pallas_call(
        _tc_layer2_body,
        grid=grid,
        in_specs=[
            pl.BlockSpec((NC, BN, 64), lambda i: (0, i, 0)),
            pl.BlockSpec((BN, 8), lambda i: (i, 0)),
            pl.BlockSpec((BN, 64), lambda i: (i, 0)),
        ],
        out_specs=pl.BlockSpec((BN, 64), lambda i: (i, 0)),
        out_shape=jax.ShapeDtypeStruct((N, 64), jnp.float32),
    )(acc2, deg8, hr)
    return out


# speedup vs baseline: 1.0369x; 1.0369x over previous
import functools

import jax
import jax.numpy as jnp
from jax import lax
from jax.experimental import pallas as pl
from jax.experimental.pallas import tpu as pltpu
from jax.experimental.pallas import tpu_sc as plsc

N = 10000
E = 320000
NC = 2
NS = 16
NW = NC * NS
CHUNK = 128
NCH_FULL = 80
NCH_LAST = 20
ROWS_PER_TILE = N // NS
BN = 1000
NBUF = 5
LOOK = NBUF - 2


def _make_sc_aggregate(D):
    mesh = plsc.VectorSubcoreMesh(core_axis_name="c", subcore_axis_name="s")
    scratch = (
        [pltpu.VMEM((NCH_FULL, CHUNK), jnp.int32),
         pltpu.VMEM((NCH_FULL, CHUNK), jnp.int32),
         pltpu.VMEM((NBUF, CHUNK, D), jnp.float32),
         pltpu.VMEM_SHARED((N, D), jnp.float32)]
        + [pltpu.SemaphoreType.DMA] * (2 * NBUF)
    )

    @functools.partial(
        pl.kernel, mesh=mesh,
        out_type=jax.ShapeDtypeStruct((NC, N, D), jnp.float32),
        scratch_types=scratch,
        compiler_params=pltpu.CompilerParams(use_tc_tiling_on_sc=False))
    def agg(vals, src3, dst3, out, src_v, dst_v, rows, acc, *sems):
        gsem = sems[:NBUF]
        ssem = sems[NBUF:]
        c = lax.axis_index("c")
        s = lax.axis_index("s")
        wid = c * NS + s
        r0 = s * ROWS_PER_TILE
        nch = jnp.where(wid == NW - 1, NCH_LAST, NCH_FULL)

        zero16 = jnp.zeros((16,), jnp.float32)

        def zrow(r, _):
            def zcol(k, _):
                rows[0, r, pl.ds(k * 16, 16)] = zero16
                return 0
            return lax.fori_loop(0, D // 16, zcol, 0)
        lax.fori_loop(0, CHUNK, zrow, 0)
        for b in range(ROWS_PER_TILE // CHUNK):
            pltpu.sync_copy(rows.at[0], acc.at[pl.ds(r0 + b * CHUNK, CHUNK)])
        rem = ROWS_PER_TILE % CHUNK
        base = ROWS_PER_TILE - rem
        pltpu.sync_copy(rows.at[0, pl.ds(0, rem)],
                        acc.at[pl.ds(r0 + base, rem)])

        @pl.when(wid == NW - 1)
        def _():
            pltpu.sync_copy(src3.at[pl.ds((NW - 1) * NCH_FULL, NCH_LAST)],
                            src_v.at[pl.ds(0, NCH_LAST)])
            pltpu.sync_copy(dst3.at[pl.ds((NW - 1) * NCH_FULL, NCH_LAST)],
                            dst_v.at[pl.ds(0, NCH_LAST)])

        @pl.when(wid != NW - 1)
        def _():
            pltpu.sync_copy(src3.at[pl.ds(wid * NCH_FULL, NCH_FULL)], src_v)
            pltpu.sync_copy(dst3.at[pl.ds(wid * NCH_FULL, NCH_FULL)], dst_v)

        plsc.subcore_barrier()

        def fire_g(j, b):
            pltpu.async_copy(vals.at[src_v.at[j]], rows.at[b], gsem[b])

        def wait_g(j, b):
            pltpu.make_async_copy(vals.at[src_v.at[j]], rows.at[b],
                                  gsem[b]).wait()

        def fire_s(j, b):
            pltpu.async_copy(rows.at[b], acc.at[dst_v.at[j]], ssem[b],
                             add=True)

        def wait_s(j, b):
            pltpu.make_async_copy(rows.at[b], acc.at[dst_v.at[j]],
                                  ssem[b]).wait()

        for j in range(LOOK):
            fire_g(j, j)
        for j in range(NBUF):
            if j - 2 >= 0:
                wait_s(j - 2, (j - 2) % NBUF)
            fire_g(j + LOOK, (j + LOOK) % NBUF)
            wait_g(j, j)
            fire_s(j, j)

        def body(i, _):
            j0 = i * NBUF
            for b in range(NBUF):
                j = j0 + b
                wait_s(j - 2, (b - 2) % NBUF)
                fire_g(j + LOOK, (b + LOOK) % NBUF)
                wait_g(j, b)
                fire_s(j, b)
            return 0
        lax.fori_loop(1, nch // NBUF - 1, body, 0)

        for k in range(NBUF):
            j = nch - NBUF + k
            wait_s(j - 2, (k - 2) % NBUF)
            if k + LOOK < NBUF:
                fire_g(j + LOOK, (k + LOOK) % NBUF)
            wait_g(j, k)
            fire_s(j, k)
        for k in range(NBUF - 2, NBUF):
            wait_s(nch - NBUF + k, k)

        plsc.subcore_barrier()

        pltpu.sync_copy(acc.at[pl.ds(r0, ROWS_PER_TILE)],
                        out.at[c, pl.ds(r0, ROWS_PER_TILE)])

    return agg


_sc_agg_80 = _make_sc_aggregate(80)
_sc_agg_64 = _make_sc_aggregate(64)


def _tc_xr_body(x_ref, w1rt_ref, b1_ref, xr_ref, xb_ref):
    x = x_ref[...]
    xr_ref[...] = x @ w1rt_ref[...] + b1_ref[...]
    nb = x.shape[0]
    xb_ref[...] = jnp.concatenate(
        [x[:, 80:], jnp.ones((nb, 1), jnp.float32),
         jnp.zeros((nb, 15), jnp.float32)], axis=1)


def _tc_layer1_body(acca_ref, accb_ref, xr_ref, w1lta_ref, w1ltb_ref,
                    w2lt_ref, y2_ref, h_ref, deg_ref):
    a = acca_ref[0] + acca_ref[1]
    b = accb_ref[0] + accb_ref[1]
    deg = jnp.maximum(b[:, 48], 1.0)
    lin = a @ w1lta_ref[...] + b[:, :48] @ w1ltb_ref[...]
    h = jnp.maximum(lin / deg[:, None] + xr_ref[...], 0.0)
    h_ref[...] = h
    y2_ref[...] = h @ w2lt_ref[...]
    deg_ref[...] = jnp.broadcast_to(deg[:, None], deg_ref.shape)


def _tc_hr_body(h_ref, w2rt_ref, b2_ref, hr_ref):
    hr_ref[...] = h_ref[...] @ w2rt_ref[...] + b2_ref[...]


def _tc_layer2_body(acc_ref, deg_ref, hr_ref, o_ref):
    a = acc_ref[0] + acc_ref[1]
    v = a / deg_ref[:, :1] + hr_ref[...]
    z = v - jnp.max(v, axis=-1, keepdims=True)
    o_ref[...] = z - jnp.log(jnp.sum(jnp.exp(z), axis=-1, keepdims=True))


def kernel(x, edge_index, W1_l, b1, W1_r, W2_l, b2, W2_r):
    ei = edge_index.astype(jnp.int32)
    src3 = ei[0].reshape(E // CHUNK, CHUNK)
    dst3 = ei[1].reshape(E // CHUNK, CHUNK)

    xa = x[:, :80]

    grid = (N // BN,)
    wfull = lambda shp: pl.BlockSpec(shp, lambda i: (0, 0))

    xr, xb = pl.pallas_call(
        _tc_xr_body,
        grid=grid,
        in_specs=[pl.BlockSpec((BN, 128), lambda i: (i, 0)),
                  wfull((128, 128)), wfull((1, 128))],
        out_specs=[pl.BlockSpec((BN, 128), lambda i: (i, 0)),
                   pl.BlockSpec((BN, 64), lambda i: (i, 0))],
        out_shape=[jax.ShapeDtypeStruct((N, 128), jnp.float32),
                   jax.ShapeDtypeStruct((N, 64), jnp.float32)],
    )(x, W1_r.T, b1[None, :])

    acca = _sc_agg_80(xa, src3, dst3)
    accb = _sc_agg_64(xb, src3, dst3)

    y2, h, deg8 = pl.pallas_call(
        _tc_layer1_body,
        grid=grid,
        in_specs=[
            pl.BlockSpec((NC, BN, 80), lambda i: (0, i, 0)),
            pl.BlockSpec((NC, BN, 64), lambda i: (0, i, 0)),
            pl.BlockSpec((BN, 128), lambda i: (i, 0)),
            wfull((80, 128)),
            wfull((48, 128)),
            wfull((128, 64)),
        ],
        out_specs=[pl.BlockSpec((BN, 64), lambda i: (i, 0)),
                   pl.BlockSpec((BN, 128), lambda i: (i, 0)),
                   pl.BlockSpec((BN, 8), lambda i: (i, 0))],
        out_shape=[jax.ShapeDtypeStruct((N, 64), jnp.float32),
                   jax.ShapeDtypeStruct((N, 128), jnp.float32),
                   jax.ShapeDtypeStruct((N, 8), jnp.float32)],
    )(acca, accb, xr, W1_l.T[:80], W1_l.T[80:], W2_l.T)

    acc2 = _sc_agg_64(y2, src3, dst3)

    hr = pl.pallas_call(
        _tc_hr_body,
        grid=grid,
        in_specs=[pl.BlockSpec((BN, 128), lambda i: (i, 0)),
                  wfull((128, 64)), wfull((1, 64))],
        out_specs=pl.BlockSpec((BN, 64), lambda i: (i, 0)),
        out_shape=jax.ShapeDtypeStruct((N, 64), jnp.float32),
    )(h, W2_r.T, b2[None, :])

    out = pl.pallas_call(
        _tc_layer2_body,
        grid=grid,
        in_specs=[
            pl.BlockSpec((NC, BN, 64), lambda i: (0, i, 0)),
            pl.BlockSpec((BN, 8), lambda i: (i, 0)),
            pl.BlockSpec((BN, 64), lambda i: (i, 0)),
        ],
        out_specs=pl.BlockSpec((BN, 64), lambda i: (i, 0)),
        out_shape=jax.ShapeDtypeStruct((N, 64), jnp.float32),
    )(acc2, deg8, hr)
    return out


# speedup vs baseline: 1.0759x; 1.0376x over previous
import functools

import jax
import jax.numpy as jnp
from jax import lax
from jax.experimental import pallas as pl
from jax.experimental.pallas import tpu as pltpu
from jax.experimental.pallas import tpu_sc as plsc

N = 10000
E = 320000
NC = 2
NS = 16
NW = NC * NS
CHUNK = 128
NCH_FULL = 80
NCH_LAST = 20
ROWS_PER_TILE = N // NS
BN = 1000
NBUF = 5
LOOK = NBUF - 2


def _make_sc_aggregate(D):
    mesh = plsc.VectorSubcoreMesh(core_axis_name="c", subcore_axis_name="s")
    scratch = (
        [pltpu.VMEM((NCH_FULL, CHUNK), jnp.int32),
         pltpu.VMEM((NCH_FULL, CHUNK), jnp.int32),
         pltpu.VMEM((NBUF, CHUNK, D), jnp.float32),
         pltpu.VMEM_SHARED((N, D), jnp.float32)]
        + [pltpu.SemaphoreType.DMA] * (2 * NBUF)
    )

    @functools.partial(
        pl.kernel, mesh=mesh,
        out_type=jax.ShapeDtypeStruct((NC, N, D), jnp.float32),
        scratch_types=scratch,
        compiler_params=pltpu.CompilerParams(use_tc_tiling_on_sc=False))
    def agg(vals, src3, dst3, out, src_v, dst_v, rows, acc, *sems):
        gsem = sems[:NBUF]
        ssem = sems[NBUF:]
        c = lax.axis_index("c")
        s = lax.axis_index("s")
        wid = c * NS + s
        r0 = s * ROWS_PER_TILE
        nch = jnp.where(wid == NW - 1, NCH_LAST, NCH_FULL)

        zero16 = jnp.zeros((16,), jnp.float32)

        def zrow(r, _):
            def zcol(k, _):
                rows[0, r, pl.ds(k * 16, 16)] = zero16
                return 0
            return lax.fori_loop(0, D // 16, zcol, 0)
        lax.fori_loop(0, CHUNK, zrow, 0)
        for b in range(ROWS_PER_TILE // CHUNK):
            pltpu.sync_copy(rows.at[0], acc.at[pl.ds(r0 + b * CHUNK, CHUNK)])
        rem = ROWS_PER_TILE % CHUNK
        base = ROWS_PER_TILE - rem
        pltpu.sync_copy(rows.at[0, pl.ds(0, rem)],
                        acc.at[pl.ds(r0 + base, rem)])

        @pl.when(wid == NW - 1)
        def _():
            pltpu.sync_copy(src3.at[pl.ds((NW - 1) * NCH_FULL, NCH_LAST)],
                            src_v.at[pl.ds(0, NCH_LAST)])
            pltpu.sync_copy(dst3.at[pl.ds((NW - 1) * NCH_FULL, NCH_LAST)],
                            dst_v.at[pl.ds(0, NCH_LAST)])

        @pl.when(wid != NW - 1)
        def _():
            pltpu.sync_copy(src3.at[pl.ds(wid * NCH_FULL, NCH_FULL)], src_v)
            pltpu.sync_copy(dst3.at[pl.ds(wid * NCH_FULL, NCH_FULL)], dst_v)

        plsc.subcore_barrier()

        def fire_g(j, b):
            pltpu.async_copy(vals.at[src_v.at[j]], rows.at[b], gsem[b])

        def wait_g(j, b):
            pltpu.make_async_copy(vals.at[src_v.at[j]], rows.at[b],
                                  gsem[b]).wait()

        def fire_s(j, b):
            pltpu.async_copy(rows.at[b], acc.at[dst_v.at[j]], ssem[b],
                             add=True)

        def wait_s(j, b):
            pltpu.make_async_copy(rows.at[b], acc.at[dst_v.at[j]],
                                  ssem[b]).wait()

        for j in range(LOOK):
            fire_g(j, j)
        for j in range(NBUF):
            if j - 2 >= 0:
                wait_s(j - 2, (j - 2) % NBUF)
            fire_g(j + LOOK, (j + LOOK) % NBUF)
            wait_g(j, j)
            fire_s(j, j)

        def body(i, _):
            j0 = i * NBUF
            for b in range(NBUF):
                j = j0 + b
                wait_s(j - 2, (b - 2) % NBUF)
                fire_g(j + LOOK, (b + LOOK) % NBUF)
                wait_g(j, b)
                fire_s(j, b)
            return 0
        lax.fori_loop(1, nch // NBUF - 1, body, 0)

        for k in range(NBUF):
            j = nch - NBUF + k
            wait_s(j - 2, (k - 2) % NBUF)
            if k + LOOK < NBUF:
                fire_g(j + LOOK, (k + LOOK) % NBUF)
            wait_g(j, k)
            fire_s(j, k)
        for k in range(NBUF - 2, NBUF):
            wait_s(nch - NBUF + k, k)

        plsc.subcore_barrier()

        pltpu.sync_copy(acc.at[pl.ds(r0, ROWS_PER_TILE)],
                        out.at[c, pl.ds(r0, ROWS_PER_TILE)])

    return agg


_sc_agg_80 = _make_sc_aggregate(80)
_sc_agg_64 = _make_sc_aggregate(64)


def _matmul_t(a, w):
    return lax.dot_general(a, w, (((1,), (1,)), ((), ())),
                           preferred_element_type=jnp.float32)


def _tc_xr_body(x_ref, w1r_ref, b1_ref, xr_ref, xb_ref):
    x = x_ref[...]
    xr_ref[...] = _matmul_t(x, w1r_ref[...]) + b1_ref[...]
    nb = x.shape[0]
    xb_ref[...] = jnp.concatenate(
        [x[:, 80:], jnp.ones((nb, 1), jnp.float32),
         jnp.zeros((nb, 15), jnp.float32)], axis=1)


def _tc_layer1_body(acca_ref, accb_ref, xr_ref, w1l_ref, w2l_ref,
                    y2_ref, h_ref, deg_ref):
    a = acca_ref[0] + acca_ref[1]
    b = accb_ref[0] + accb_ref[1]
    deg = jnp.maximum(b[:, 48], 1.0)
    w1l = w1l_ref[...]
    lin = _matmul_t(a, w1l[:, :80]) + _matmul_t(b[:, :48], w1l[:, 80:])
    h = jnp.maximum(lin / deg[:, None] + xr_ref[...], 0.0)
    h_ref[...] = h
    y2_ref[...] = _matmul_t(h, w2l_ref[...])
    deg_ref[...] = jnp.broadcast_to(deg[:, None], deg_ref.shape)


def _tc_hr_body(h_ref, w2r_ref, b2_ref, hr_ref):
    hr_ref[...] = _matmul_t(h_ref[...], w2r_ref[...]) + b2_ref[...]


def _tc_layer2_body(acc_ref, deg_ref, hr_ref, o_ref):
    a = acc_ref[0] + acc_ref[1]
    v = a / deg_ref[:, :1] + hr_ref[...]
    z = v - jnp.max(v, axis=-1, keepdims=True)
    o_ref[...] = z - jnp.log(jnp.sum(jnp.exp(z), axis=-1, keepdims=True))


def kernel(x, edge_index, W1_l, b1, W1_r, W2_l, b2, W2_r):
    ei = edge_index.astype(jnp.int32)
    src3 = ei[0].reshape(E // CHUNK, CHUNK)
    dst3 = ei[1].reshape(E // CHUNK, CHUNK)

    xa = x[:, :80]

    grid = (N // BN,)
    wfull = lambda shp: pl.BlockSpec(shp, lambda i: (0, 0))

    xr, xb = pl.pallas_call(
        _tc_xr_body,
        grid=grid,
        in_specs=[pl.BlockSpec((BN, 128), lambda i: (i, 0)),
                  wfull((128, 128)), wfull((1, 128))],
        out_specs=[pl.BlockSpec((BN, 128), lambda i: (i, 0)),
                   pl.BlockSpec((BN, 64), lambda i: (i, 0))],
        out_shape=[jax.ShapeDtypeStruct((N, 128), jnp.float32),
                   jax.ShapeDtypeStruct((N, 64), jnp.float32)],
    )(x, W1_r, b1[None, :])

    acca = _sc_agg_80(xa, src3, dst3)
    accb = _sc_agg_64(xb, src3, dst3)

    y2, h, deg8 = pl.pallas_call(
        _tc_layer1_body,
        grid=grid,
        in_specs=[
            pl.BlockSpec((NC, BN, 80), lambda i: (0, i, 0)),
            pl.BlockSpec((NC, BN, 64), lambda i: (0, i, 0)),
            pl.BlockSpec((BN, 128), lambda i: (i, 0)),
            wfull((128, 128)),
            wfull((64, 128)),
        ],
        out_specs=[pl.BlockSpec((BN, 64), lambda i: (i, 0)),
                   pl.BlockSpec((BN, 128), lambda i: (i, 0)),
                   pl.BlockSpec((BN, 8), lambda i: (i, 0))],
        out_shape=[jax.ShapeDtypeStruct((N, 64), jnp.float32),
                   jax.ShapeDtypeStruct((N, 128), jnp.float32),
                   jax.ShapeDtypeStruct((N, 8), jnp.float32)],
    )(acca, accb, xr, W1_l, W2_l)

    acc2 = _sc_agg_64(y2, src3, dst3)

    hr = pl.pallas_call(
        _tc_hr_body,
        grid=grid,
        in_specs=[pl.BlockSpec((BN, 128), lambda i: (i, 0)),
                  wfull((64, 128)), wfull((1, 64))],
        out_specs=pl.BlockSpec((BN, 64), lambda i: (i, 0)),
        out_shape=jax.ShapeDtypeStruct((N, 64), jnp.float32),
    )(h, W2_r, b2[None, :])

    out = pl.pallas_call(
        _tc_layer2_body,
        grid=grid,
        in_specs=[
            pl.BlockSpec((NC, BN, 64), lambda i: (0, i, 0)),
            pl.BlockSpec((BN, 8), lambda i: (i, 0)),
            pl.BlockSpec((BN, 64), lambda i: (i, 0)),
        ],
        out_specs=pl.BlockSpec((BN, 64), lambda i: (i, 0)),
        out_shape=jax.ShapeDtypeStruct((N, 64), jnp.float32),
    )(acc2, deg8, hr)
    return out
